# Initial kernel scaffold; baseline (speedup 1.0000x reference)
import jax
import jax.numpy as jnp
from jax.experimental import pallas as pl


def kernel(inputs, embeddings):
    raise NotImplementedError("write your pallas kernel here")


# speedup vs baseline: 1.5681x; 1.5681x over previous
import functools

import jax
import jax.numpy as jnp
from jax import lax
from jax.experimental import pallas as pl
from jax.experimental.pallas import tpu as pltpu
from jax.experimental.pallas import tpu_sc as plsc

BATCH = 16384
FIELDS = 26
D = 32
NC = 2
NS = 16
NW = NC * NS
TOTAL = BATCH * FIELDS
PER_W = TOTAL // NW
CHUNK = 128
NCHUNK = PER_W // CHUNK
NBUF = 4


def _sc_gather(idx, table):
    mesh = plsc.VectorSubcoreMesh(core_axis_name="c", subcore_axis_name="s")

    @functools.partial(
        pl.kernel,
        mesh=mesh,
        out_type=jax.ShapeDtypeStruct((TOTAL, D), jnp.float32),
        compiler_params=pltpu.CompilerParams(use_tc_tiling_on_sc=False),
        scratch_types=[
            pltpu.VMEM((NCHUNK, CHUNK), jnp.int32),
            *[pltpu.VMEM((CHUNK, D), jnp.float32) for _ in range(NBUF)],
            *[pltpu.SemaphoreType.DMA for _ in range(NBUF)],
        ],
    )
    def k(idx_hbm, table_hbm, out_hbm, idx_v, b0, b1, b2, b3, s0, s1, s2, s3):
        bufs = (b0, b1, b2, b3)
        sems = (s0, s1, s2, s3)
        wid = lax.axis_index("s") * NC + lax.axis_index("c")
        base = wid * PER_W
        pltpu.sync_copy(idx_hbm.at[wid], idx_v)
        for b in range(NBUF):
            pltpu.async_copy(table_hbm.at[idx_v.at[b]], bufs[b], sems[b])

        def body(g, carry):
            for b in range(NBUF):
                j = g * NBUF + b
                pltpu.make_async_copy(
                    table_hbm.at[idx_v.at[0]], bufs[b], sems[b]
                ).wait()
                pltpu.sync_copy(
                    bufs[b], out_hbm.at[pl.ds(base + j * CHUNK, CHUNK)]
                )
                jn = j + NBUF

                @pl.when(jn < NCHUNK)
                def _():
                    pltpu.async_copy(table_hbm.at[idx_v.at[jn]], bufs[b], sems[b])
            return carry

        lax.fori_loop(0, NCHUNK // NBUF, body, 0)

    return k(idx, table)


def kernel(inputs, embeddings):
    idx = inputs.reshape(NW, NCHUNK, CHUNK)
    out = _sc_gather(idx, embeddings)
    return out.reshape(BATCH, FIELDS, D)


# speedup vs baseline: 1.5779x; 1.0062x over previous
import functools

import jax
import jax.numpy as jnp
from jax import lax
from jax.experimental import pallas as pl
from jax.experimental.pallas import tpu as pltpu
from jax.experimental.pallas import tpu_sc as plsc

BATCH = 16384
FIELDS = 26
D = 32
NC = 2
NS = 16
NW = NC * NS
TOTAL = BATCH * FIELDS
PER_W = TOTAL // NW
CHUNK = 128
NCHUNK = PER_W // CHUNK
NBUF = 8


def _sc_gather(idx, table):
    mesh = plsc.VectorSubcoreMesh(core_axis_name="c", subcore_axis_name="s")

    @functools.partial(
        pl.kernel,
        mesh=mesh,
        out_type=jax.ShapeDtypeStruct((TOTAL, D), jnp.float32),
        compiler_params=pltpu.CompilerParams(use_tc_tiling_on_sc=False),
        scratch_types=[
            pltpu.VMEM((NCHUNK, CHUNK), jnp.int32),
            *[pltpu.VMEM((CHUNK, D), jnp.float32) for _ in range(NBUF)],
            *[pltpu.SemaphoreType.DMA for _ in range(NBUF)],
        ],
    )
    def k(idx_hbm, table_hbm, out_hbm, idx_v, *scr):
        bufs = scr[:NBUF]
        sems = scr[NBUF:]
        wid = lax.axis_index("s") * NC + lax.axis_index("c")
        base = wid * PER_W
        pltpu.sync_copy(idx_hbm.at[wid], idx_v)
        for b in range(NBUF):
            pltpu.async_copy(table_hbm.at[idx_v.at[b]], bufs[b], sems[b])

        def body(g, carry):
            for b in range(NBUF):
                j = g * NBUF + b
                pltpu.make_async_copy(
                    table_hbm.at[idx_v.at[0]], bufs[b], sems[b]
                ).wait()
                pltpu.sync_copy(
                    bufs[b], out_hbm.at[pl.ds(base + j * CHUNK, CHUNK)]
                )
                jn = j + NBUF

                @pl.when(jn < NCHUNK)
                def _():
                    pltpu.async_copy(table_hbm.at[idx_v.at[jn]], bufs[b], sems[b])
            return carry

        lax.fori_loop(0, NCHUNK // NBUF, body, 0)

    return k(idx, table)


def kernel(inputs, embeddings):
    idx = inputs.reshape(NW, NCHUNK, CHUNK)
    out = _sc_gather(idx, embeddings)
    return out.reshape(BATCH, FIELDS, D)
